# baseline (speedup 1.0000x reference)
import jax
import jax.numpy as jnp
from jax.experimental import pallas as pl


def _add_body(x_ref, pos_ref, o_ref):
    o_ref[...] = x_ref[...] + pos_ref[...][None]


def kernel(patch, pos_table):
    B, P, D = patch.shape
    xt = jnp.transpose(patch, (0, 2, 1))
    post = jnp.transpose(pos_table, (1, 0))
    BB = 4
    out_t = pl.pallas_call(
        _add_body,
        grid=(B // BB,),
        in_specs=[
            pl.BlockSpec((BB, D, P), lambda i: (i, 0, 0)),
            pl.BlockSpec((D, P), lambda i: (0, 0)),
        ],
        out_specs=pl.BlockSpec((BB, D, P), lambda i: (i, 0, 0)),
        out_shape=jax.ShapeDtypeStruct((B, D, P), jnp.float32),
    )(xt, post)
    return jnp.transpose(out_t, (0, 2, 1))


# speedup vs baseline: 1.2499x; 1.2499x over previous
import jax
import jax.numpy as jnp
from jax.experimental import pallas as pl


def _add_body(x_ref, pos_ref, o_ref):
    o_ref[...] = x_ref[...] + pos_ref[...][None]


def kernel(patch, pos_table):
    B, P, D = patch.shape
    xt = jnp.transpose(patch, (0, 2, 1))
    post = jnp.transpose(pos_table, (1, 0))
    BB = 16
    out_t = pl.pallas_call(
        _add_body,
        grid=(B // BB,),
        in_specs=[
            pl.BlockSpec((BB, D, P), lambda i: (i, 0, 0)),
            pl.BlockSpec((D, P), lambda i: (0, 0)),
        ],
        out_specs=pl.BlockSpec((BB, D, P), lambda i: (i, 0, 0)),
        out_shape=jax.ShapeDtypeStruct((B, D, P), jnp.float32),
    )(xt, post)
    return jnp.transpose(out_t, (0, 2, 1))


# speedup vs baseline: 1.2971x; 1.0377x over previous
import jax
import jax.numpy as jnp
from jax.experimental import pallas as pl


def _add_body(x_ref, pos_ref, o_ref):
    o_ref[...] = x_ref[...] + pos_ref[...][None]


def kernel(patch, pos_table):
    B, P, D = patch.shape
    xt = jnp.transpose(patch, (0, 2, 1))
    post = jnp.transpose(pos_table, (1, 0))
    BB = 32
    out_t = pl.pallas_call(
        _add_body,
        grid=(B // BB,),
        in_specs=[
            pl.BlockSpec((BB, D, P), lambda i: (i, 0, 0)),
            pl.BlockSpec((D, P), lambda i: (0, 0)),
        ],
        out_specs=pl.BlockSpec((BB, D, P), lambda i: (i, 0, 0)),
        out_shape=jax.ShapeDtypeStruct((B, D, P), jnp.float32),
    )(xt, post)
    return jnp.transpose(out_t, (0, 2, 1))
